# Initial kernel scaffold; baseline (speedup 1.0000x reference)
import jax
import jax.numpy as jnp
from jax.experimental import pallas as pl


def kernel(user_table, entity_table, W1_0, b1_0, W2_0, b2_0, W1_1, b1_1, W2_1, b2_1, edge_vals, edge_index, user_ids, item_ids):
    raise NotImplementedError("write your pallas kernel here")


# speedup vs baseline: 2.7213x; 2.7213x over previous
import functools

import jax
import jax.numpy as jnp
from jax import lax
from jax.experimental import pallas as pl
from jax.experimental.pallas import tpu as pltpu
from jax.experimental.pallas import tpu_sc as plsc

N_USERS = 10000
N_ENT = 40000
NN = N_USERS + N_ENT
E = 800000
D = 64

NC = 2
NS = 16
HALF = NN // 2
ROWS_PER_TILE = 1600
ACC_ROWS = NS * ROWS_PER_TILE
TRASH = ACC_ROWS - 1

K = 128
CHUNKS_PER_TILE = 392
PER_TILE = CHUNKS_PER_TILE * K
E_PAD = NS * PER_TILE
C2 = CHUNKS_PER_TILE // 2

WB = 40

_mesh = plsc.VectorSubcoreMesh(core_axis_name="c", subcore_axis_name="s")


def _zero16():
    return jnp.zeros((16,), jnp.float32)


@functools.partial(
    pl.kernel,
    out_type=jax.ShapeDtypeStruct((NN, D), jnp.float32),
    mesh=_mesh,
    scratch_types=[
        pltpu.VMEM((K,), jnp.int32),
        pltpu.VMEM((K,), jnp.int32),
        pltpu.VMEM((K,), jnp.int32),
        pltpu.VMEM((K,), jnp.int32),
        pltpu.VMEM((K,), jnp.float32),
        pltpu.VMEM((K,), jnp.float32),
        pltpu.VMEM((K,), jnp.int32),
        pltpu.VMEM((K, 4 * D), jnp.uint8),
        pltpu.VMEM((K, 4 * D), jnp.uint8),
        pltpu.VMEM((K, D), jnp.float32),
        pltpu.VMEM((WB, D), jnp.float32),
        pltpu.VMEM_SHARED((ACC_ROWS, D), jnp.float32),
        pltpu.SemaphoreType.DMA,
        pltpu.SemaphoreType.DMA,
    ],
    compiler_params=pltpu.CompilerParams(use_tc_tiling_on_sc=False, needs_layout_passes=False),
)
def _spmm(src_hbm, dst_hbm, val_hbm, feats_hbm, out_hbm,
          srcb0, srcb1, dstl0, dstl1, valb0, valb1, dstb,
          rows0, rows1, rowsf, zwbuf, acc, sem0, sem1):
    c = lax.axis_index("c")
    s = lax.axis_index("s")
    lo = c * HALF
    tile_base = s * PER_TILE

    def _fill_z(r, _):
        for j in range(D // 16):
            zwbuf[r, pl.ds(j * 16, 16)] = _zero16()
        return 0
    lax.fori_loop(0, WB, _fill_z, 0)

    def _zero_acc(k, _):
        pltpu.sync_copy(zwbuf, acc.at[pl.ds(s * ROWS_PER_TILE + k * WB, WB)])
        return 0
    lax.fori_loop(0, ROWS_PER_TILE // WB, _zero_acc, 0)
    plsc.subcore_barrier()

    def _fire(ci, srcb, dstl, valb, rows, sem):
        base = tile_base + ci * K
        pltpu.sync_copy(src_hbm.at[pl.ds(base, K)], srcb)
        pltpu.sync_copy(dst_hbm.at[pl.ds(base, K)], dstb)
        pltpu.sync_copy(val_hbm.at[pl.ds(base, K)], valb)
        for v in range(K // 16):
            sl = pl.ds(v * 16, 16)
            dl = dstb[sl] - lo
            inm = (dl >= 0) & (dl < HALF)
            dstl[sl] = jnp.where(inm, dl, TRASH)
            valb[sl] = jnp.where(inm, valb[sl], 0.0)
        pltpu.async_copy(feats_hbm.at[srcb], rows, sem)

    def _drain(srcb, rows, sem):
        pltpu.make_async_copy(feats_hbm.at[srcb], rows, sem).wait()

    def _scale(valb, rows):
        def _e16(i, _):
            vv = valb[pl.ds(i * 16, 16)]
            for de in range(16):
                e = i * 16 + de
                v = lax.gather(
                    vv, jnp.full((16, 1), de, jnp.int32),
                    lax.GatherDimensionNumbers(
                        offset_dims=(), collapsed_slice_dims=(0,),
                        start_index_map=(0,)),
                    slice_sizes=(1,),
                    mode=lax.GatherScatterMode.PROMISE_IN_BOUNDS)
                for j in range(D // 16):
                    r = plsc.bitcast(rows[e, pl.ds(j * 64, 64)], jnp.float32)
                    rowsf[e, pl.ds(j * 16, 16)] = r * v
            return 0
        lax.fori_loop(0, K // 16, _e16, 0)

    _fire(0, srcb0, dstl0, valb0, rows0, sem0)

    def _pair(p, _):
        _fire(2 * p + 1, srcb1, dstl1, valb1, rows1, sem1)
        _drain(srcb0, rows0, sem0)
        _scale(valb0, rows0)
        pltpu.sync_copy(rowsf, acc.at[dstl0], add=True)

        @pl.when(p + 1 < C2)
        def _():
            _fire(2 * p + 2, srcb0, dstl0, valb0, rows0, sem0)

        _drain(srcb1, rows1, sem1)
        _scale(valb1, rows1)
        pltpu.sync_copy(rowsf, acc.at[dstl1], add=True)
        return 0
    lax.fori_loop(0, C2, _pair, 0)
    plsc.subcore_barrier()

    def _wb(k, _):
        row0 = s * ROWS_PER_TILE + k * WB

        @pl.when(row0 < HALF)
        def _():
            pltpu.sync_copy(acc.at[pl.ds(row0, WB)], zwbuf)
            pltpu.sync_copy(zwbuf, out_hbm.at[pl.ds(lo + row0, WB)])
        return 0
    lax.fori_loop(0, ROWS_PER_TILE // WB, _wb, 0)


def _dense_layer(x, h, W1, W2, b, dout):
    RB = 400
    nblk = NN // RB

    def body(x_ref, h_ref, w1_ref, w2_ref, b_ref, o_ref):
        xv = x_ref[...]
        hv = h_ref[...]
        t = ((xv + hv) @ w1_ref[...] + (xv * hv) @ w2_ref[...]
             + b_ref[...])
        t = jnp.where(t >= 0, t, 0.01 * t)
        n = jnp.sqrt(jnp.sum(t * t, axis=1, keepdims=True))
        o_ref[...] = t / jnp.maximum(n, 1e-12)

    din = x.shape[1]
    return pl.pallas_call(
        body,
        grid=(nblk,),
        in_specs=[
            pl.BlockSpec((RB, din), lambda i: (i, 0)),
            pl.BlockSpec((RB, din), lambda i: (i, 0)),
            pl.BlockSpec((din, dout), lambda i: (0, 0)),
            pl.BlockSpec((din, dout), lambda i: (0, 0)),
            pl.BlockSpec((1, dout), lambda i: (0, 0)),
        ],
        out_specs=pl.BlockSpec((RB, dout), lambda i: (i, 0)),
        out_shape=jax.ShapeDtypeStruct((NN, dout), jnp.float32),
    )(x, h, W1, W2, b)


_B = 4096
_PT = _B // (NC * NS)


@functools.partial(
    pl.kernel,
    out_type=(
        jax.ShapeDtypeStruct((_B, 4 * D), jnp.uint8),
        jax.ShapeDtypeStruct((_B, 4 * D), jnp.uint8),
        jax.ShapeDtypeStruct((_B, 4 * D), jnp.uint8),
        jax.ShapeDtypeStruct((_B, 4 * D), jnp.uint8),
        jax.ShapeDtypeStruct((_B, 128), jnp.uint8),
        jax.ShapeDtypeStruct((_B, 128), jnp.uint8),
    ),
    mesh=_mesh,
    scratch_types=[
        pltpu.VMEM((_PT,), jnp.int32),
        pltpu.VMEM((_PT,), jnp.int32),
        pltpu.VMEM((_PT, 4 * D), jnp.uint8),
        pltpu.VMEM((_PT, 128), jnp.uint8),
        pltpu.SemaphoreType.DMA,
    ],
    compiler_params=pltpu.CompilerParams(use_tc_tiling_on_sc=False, needs_layout_passes=False),
)
def _gather6(x0, h1, h2, uid, iid, u0o, i0o, u1o, i1o, u2o, i2o,
             uidx, iidx, b64, b32, sem):
    wid = lax.axis_index("s") * NC + lax.axis_index("c")
    base = wid * _PT
    pltpu.sync_copy(uid.at[pl.ds(base, _PT)], uidx)
    pltpu.sync_copy(iid.at[pl.ds(base, _PT)], iidx)
    for tbl, idx, out in ((x0, uidx, u0o), (x0, iidx, i0o),
                          (h1, uidx, u1o), (h1, iidx, i1o)):
        pltpu.async_copy(tbl.at[idx], b64, sem).wait()
        pltpu.sync_copy(b64, out.at[pl.ds(base, _PT)])
    for idx, out in ((uidx, u2o), (iidx, i2o)):
        pltpu.async_copy(h2.at[idx], b32, sem).wait()
        pltpu.sync_copy(b32, out.at[pl.ds(base, _PT)])


def _dot_scores(u0, i0, u1, i1, u2, i2):
    def body(u0r, i0r, u1r, i1r, u2r, i2r, o_ref):
        sc = (jnp.sum(u0r[...] * i0r[...], axis=1)
              + jnp.sum(u1r[...] * i1r[...], axis=1)
              + jnp.sum(u2r[...] * i2r[...], axis=1))
        o_ref[...] = sc[:, None]

    out = pl.pallas_call(
        body,
        out_shape=jax.ShapeDtypeStruct((_B, 1), jnp.float32),
    )(u0, i0, u1, i1, u2, i2)
    return out.reshape(_B)


def _as_u8(x):
    n, d = x.shape
    return lax.bitcast_convert_type(x, jnp.uint8).reshape(n, d * 4)


def _as_f32(x):
    n, d = x.shape
    return lax.bitcast_convert_type(x.reshape(n, d // 4, 4), jnp.float32)


def kernel(user_table, entity_table, W1_0, b1_0, W2_0, b2_0,
           W1_1, b1_1, W2_1, b2_1, edge_vals, edge_index,
           user_ids, item_ids):
    x0 = jnp.concatenate([user_table, entity_table], axis=0)

    src = edge_index[1].astype(jnp.int32)
    dst = edge_index[0].astype(jnp.int32)
    pad = E_PAD - E
    src_p = jnp.concatenate([src, jnp.zeros((pad,), jnp.int32)])
    dst_p = jnp.concatenate([dst, jnp.full((pad,), NN, jnp.int32)])
    val_p = jnp.concatenate([edge_vals, jnp.zeros((pad,), jnp.float32)])

    nh0 = _spmm(src_p, dst_p, val_p, _as_u8(x0))
    h1 = _dense_layer(x0, nh0, W1_0, W2_0, (b1_0 + b2_0).reshape(1, -1), D)
    nh1 = _spmm(src_p, dst_p, val_p, _as_u8(h1))
    h2 = _dense_layer(h1, nh1, W1_1, W2_1, (b1_1 + b2_1).reshape(1, -1), 32)

    uid = user_ids.astype(jnp.int32)
    iid = (item_ids + N_USERS).astype(jnp.int32)
    outs = _gather6(_as_u8(x0), _as_u8(h1), _as_u8(h2), uid, iid)
    u0, i0, u1, i1, u2, i2 = (_as_f32(o) for o in outs)
    return _dot_scores(u0, i0, u1, i1, u2, i2)
